# baseline (speedup 1.0000x reference)
import functools

import jax
import jax.numpy as jnp
from jax import lax
from jax.experimental import pallas as pl
from jax.experimental.pallas import tpu as pltpu
from jax.experimental.pallas import tpu_sc as plsc

N_TOK = 16384
HID = 4096
NE = 8
TOPK = 2
HTOK = N_TOK // 2

BT = 512


def _logits_body(xa_ref, xb_ref, w_ref, *out_refs):
    dims = (((1,), (1,)), ((), ()))
    blka_t = lax.dot_general(
        xa_ref[...], w_ref[...], dims, preferred_element_type=jnp.float32
    ).T
    blkb_t = lax.dot_general(
        xb_ref[...], w_ref[...], dims, preferred_element_type=jnp.float32
    ).T
    for e in range(NE):
        out_refs[e][pl.ds(0, BT)] = blka_t[e : e + 1, :].reshape(BT)
        out_refs[e][pl.ds(BT, BT)] = blkb_t[e : e + 1, :].reshape(BT)


def _logits_tc(x, w, half):
    base = half * (HTOK // BT)
    return pl.pallas_call(
        _logits_body,
        grid=(HTOK // (2 * BT),),
        in_specs=[
            pl.BlockSpec((BT, HID), lambda i: (base + 2 * i, 0)),
            pl.BlockSpec((BT, HID), lambda i: (base + 2 * i + 1, 0)),
            pl.BlockSpec((NE, HID), lambda i: (0, 0)),
        ],
        out_specs=[
            pl.BlockSpec((2 * BT,), lambda i: (i,)) for _ in range(NE)
        ],
        out_shape=[
            jax.ShapeDtypeStruct((HTOK,), jnp.float32) for _ in range(NE)
        ],
    )(x, x, w)


NSUB = 16
TPW = HTOK // NSUB
NCH = TPW // 16
UNROLL = 8


def _make_sc_body(with_prev):
    def body(*refs):
        logit_hbm = refs[:NE]
        off = NE + (1 if with_prev else 0)
        prev_hbm = refs[NE] if with_prev else None
        s1_hbm, s2_hbm, i1_hbm, i2_hbm, cnt_hbm = refs[off : off + 5]
        lg_v, s1_v, s2_v, i1_v, i2_v, cnt_v, gat_v, part_sh, sem = \
            refs[off + 5 :]

        wid = lax.axis_index("s")
        base = wid * TPW
        copies = [
            pltpu.make_async_copy(
                logit_hbm[e].at[pl.ds(base, TPW)],
                lg_v.at[pl.ds(e * TPW, TPW)],
                sem,
            )
            for e in range(NE)
        ]
        for c in copies:
            c.start()
        for c in copies:
            c.wait()

        lanes = lax.iota(jnp.int32, 16)

        def chunk(tok0, pk):
            m1 = lg_v[pl.ds(tok0, 16)]
            i1 = jnp.zeros((16,), jnp.int32)
            m2 = jnp.full((16,), -jnp.inf, jnp.float32)
            i2 = jnp.zeros((16,), jnp.int32)
            for e in range(1, NE):
                v = lg_v[pl.ds(e * TPW + tok0, 16)]
                gt1 = v > m1
                gt2 = jnp.logical_and(jnp.logical_not(gt1), v > m2)
                m2 = jnp.where(gt1, m1, jnp.where(gt2, v, m2))
                i2 = jnp.where(gt1, i1, jnp.where(gt2, e, i2))
                m1 = jnp.where(gt1, v, m1)
                i1 = jnp.where(gt1, e, i1)
            d = jnp.exp(m2 - m1)
            r = 1.0 / (1.0 + d)
            sl = pl.ds(tok0, 16)
            s1_v[sl] = r
            s2_v[sl] = d * r
            i1_v[sl] = i1
            i2_v[sl] = i2
            one = jnp.int32(1)
            pk = pk + (one << (i1 * 4)) + (one << (i2 * 4))
            return pk

        def group(g, accs):
            pk = jnp.zeros((16,), jnp.int32)
            for j in range(UNROLL):
                pk = chunk(g * (16 * UNROLL) + j * 16, pk)
            return tuple(
                accs[e] + ((pk >> (4 * e)) & 0xF).astype(jnp.float32)
                for e in range(NE)
            )

        acc0 = tuple(jnp.zeros((16,), jnp.float32) for _ in range(NE))
        accs = lax.fori_loop(0, NCH // UNROLL, group, acc0)

        out_copies = [
            pltpu.make_async_copy(v, h.at[pl.ds(base, TPW)], sem)
            for v, h in (
                (s1_v, s1_hbm), (s2_v, s2_hbm), (i1_v, i1_hbm), (i2_v, i2_hbm)
            )
        ]
        for c in out_copies:
            c.start()
        for c in out_copies:
            c.wait()

        cv = jnp.zeros((16,), jnp.float32)
        for e in range(NE):
            cv = jnp.where(lanes == e, jnp.sum(accs[e]), cv)
        cnt_v[...] = cv
        pltpu.sync_copy(cnt_v, part_sh.at[pl.ds(wid * 16, 16)])
        plsc.subcore_barrier()

        @pl.when(wid == 0)
        def _():
            pltpu.sync_copy(part_sh, gat_v)
            tot = gat_v[pl.ds(0, 16)]
            for w in range(1, NSUB):
                tot = tot + gat_v[pl.ds(w * 16, 16)]
            if with_prev:
                pltpu.sync_copy(prev_hbm, cnt_v)
                tot = tot + cnt_v[...]
            cnt_v[...] = tot
            pltpu.sync_copy(cnt_v, cnt_hbm)

    return body


@functools.lru_cache(maxsize=2)
def _get_sc_route(with_prev):
    mesh = plsc.VectorSubcoreMesh(
        core_axis_name="c", subcore_axis_name="s",
        num_cores=1, num_subcores=NSUB,
    )
    return pl.kernel(
        _make_sc_body(with_prev),
        out_type=(
            jax.ShapeDtypeStruct((HTOK,), jnp.float32),
            jax.ShapeDtypeStruct((HTOK,), jnp.float32),
            jax.ShapeDtypeStruct((HTOK,), jnp.int32),
            jax.ShapeDtypeStruct((HTOK,), jnp.int32),
            jax.ShapeDtypeStruct((16,), jnp.float32),
        ),
        mesh=mesh,
        scratch_types=(
            pltpu.VMEM((NE * TPW,), jnp.float32),
            pltpu.VMEM((TPW,), jnp.float32),
            pltpu.VMEM((TPW,), jnp.float32),
            pltpu.VMEM((TPW,), jnp.int32),
            pltpu.VMEM((TPW,), jnp.int32),
            pltpu.VMEM((16,), jnp.float32),
            pltpu.VMEM((NSUB * 16,), jnp.float32),
            pltpu.VMEM_SHARED((NSUB * 16,), jnp.float32),
            pltpu.SemaphoreType.DMA,
        ),
        compiler_params=pltpu.CompilerParams(needs_layout_passes=False),
    )


def kernel(input, W):
    x = input.reshape(N_TOK, HID)
    la = _logits_tc(x, W, 0)
    lb = _logits_tc(x, W, 1)
    s1a, s2a, i1a, i2a, cnt_a = _get_sc_route(False)(*la)
    s1b, s2b, i1b, i2b, cnt_b = _get_sc_route(True)(*lb, cnt_a)
    s1 = jnp.concatenate([s1a, s1b])
    s2 = jnp.concatenate([s2a, s2b])
    i1 = jnp.concatenate([i1a, i1b])
    i2 = jnp.concatenate([i2a, i2b])
    return (
        jnp.stack([s1, s2], axis=1),
        jnp.stack([i1, i2], axis=1),
        cnt_b[:NE],
    )


# speedup vs baseline: 1.0306x; 1.0306x over previous
import functools

import jax
import jax.numpy as jnp
from jax import lax
from jax.experimental import pallas as pl
from jax.experimental.pallas import tpu as pltpu
from jax.experimental.pallas import tpu_sc as plsc

N_TOK = 16384
HID = 4096
NE = 8
TOPK = 2

BT = 512


def _logits_body(xa_ref, xb_ref, w_ref, *out_refs):
    dims = (((1,), (1,)), ((), ()))
    blka_t = lax.dot_general(
        xa_ref[...], w_ref[...], dims, preferred_element_type=jnp.float32
    ).T
    blkb_t = lax.dot_general(
        xb_ref[...], w_ref[...], dims, preferred_element_type=jnp.float32
    ).T
    for e in range(NE):
        out_refs[e][pl.ds(0, BT)] = blka_t[e : e + 1, :].reshape(BT)
        out_refs[e][pl.ds(BT, BT)] = blkb_t[e : e + 1, :].reshape(BT)


def _logits_tc(x, w):
    return pl.pallas_call(
        _logits_body,
        grid=(N_TOK // (2 * BT),),
        in_specs=[
            pl.BlockSpec((BT, HID), lambda i: (2 * i, 0)),
            pl.BlockSpec((BT, HID), lambda i: (2 * i + 1, 0)),
            pl.BlockSpec((NE, HID), lambda i: (0, 0)),
        ],
        out_specs=[
            pl.BlockSpec((2 * BT,), lambda i: (i,)) for _ in range(NE)
        ],
        out_shape=[
            jax.ShapeDtypeStruct((N_TOK,), jnp.float32) for _ in range(NE)
        ],
    )(x, x, w)


NSUB = 16
TPW = N_TOK // NSUB
NCH = TPW // 16
UNROLL = 8


def _sc_route_body(*refs):
    logit_hbm = refs[:NE]
    s1_hbm, s2_hbm, i1_hbm, i2_hbm, cnt_hbm = refs[NE : NE + 5]
    lg_v, s1_v, s2_v, i1_v, i2_v, cnt_v, gat_v, part_sh, sem = refs[NE + 5 :]

    wid = lax.axis_index("s")
    base = wid * TPW
    copies = [
        pltpu.make_async_copy(
            logit_hbm[e].at[pl.ds(base, TPW)],
            lg_v.at[pl.ds(e * TPW, TPW)],
            sem,
        )
        for e in range(NE)
    ]
    for c in copies:
        c.start()
    for c in copies:
        c.wait()

    lanes = lax.iota(jnp.int32, 16)

    def chunk(tok0, pk):
        m1 = lg_v[pl.ds(tok0, 16)]
        i1 = jnp.zeros((16,), jnp.int32)
        m2 = jnp.full((16,), -jnp.inf, jnp.float32)
        i2 = jnp.zeros((16,), jnp.int32)
        for e in range(1, NE):
            v = lg_v[pl.ds(e * TPW + tok0, 16)]
            gt1 = v > m1
            gt2 = jnp.logical_and(jnp.logical_not(gt1), v > m2)
            m2 = jnp.where(gt1, m1, jnp.where(gt2, v, m2))
            i2 = jnp.where(gt1, i1, jnp.where(gt2, e, i2))
            m1 = jnp.where(gt1, v, m1)
            i1 = jnp.where(gt1, e, i1)
        d = jnp.exp(m2 - m1)
        r = 1.0 / (1.0 + d)
        sl = pl.ds(tok0, 16)
        s1_v[sl] = r
        s2_v[sl] = d * r
        i1_v[sl] = i1
        i2_v[sl] = i2
        one = jnp.int32(1)
        pk = pk + (one << (i1 * 4)) + (one << (i2 * 4))
        return pk

    def group(g, accs):
        pk = jnp.zeros((16,), jnp.int32)
        for j in range(UNROLL):
            pk = chunk(g * (16 * UNROLL) + j * 16, pk)
        return tuple(
            accs[e] + ((pk >> (4 * e)) & 0xF).astype(jnp.float32)
            for e in range(NE)
        )

    acc0 = tuple(jnp.zeros((16,), jnp.float32) for _ in range(NE))
    accs = lax.fori_loop(0, NCH // UNROLL, group, acc0)

    out_copies = [
        pltpu.make_async_copy(v, h.at[pl.ds(base, TPW)], sem)
        for v, h in (
            (s1_v, s1_hbm), (s2_v, s2_hbm), (i1_v, i1_hbm), (i2_v, i2_hbm)
        )
    ]
    for c in out_copies:
        c.start()

    cv = jnp.zeros((16,), jnp.float32)
    for e in range(NE):
        cv = jnp.where(lanes == e, jnp.sum(accs[e]), cv)
    cnt_v[...] = cv
    pltpu.sync_copy(cnt_v, part_sh.at[pl.ds(wid * 16, 16)])

    for c in out_copies:
        c.wait()
    plsc.subcore_barrier()

    @pl.when(wid == 0)
    def _():
        pltpu.sync_copy(part_sh, gat_v)
        tot = gat_v[pl.ds(0, 16)]
        for w in range(1, NSUB):
            tot = tot + gat_v[pl.ds(w * 16, 16)]
        cnt_v[...] = tot
        pltpu.sync_copy(cnt_v, cnt_hbm)


@functools.lru_cache(maxsize=1)
def _get_sc_route():
    mesh = plsc.VectorSubcoreMesh(
        core_axis_name="c", subcore_axis_name="s",
        num_cores=1, num_subcores=NSUB,
    )
    return pl.kernel(
        _sc_route_body,
        out_type=(
            jax.ShapeDtypeStruct((N_TOK,), jnp.float32),
            jax.ShapeDtypeStruct((N_TOK,), jnp.float32),
            jax.ShapeDtypeStruct((N_TOK,), jnp.int32),
            jax.ShapeDtypeStruct((N_TOK,), jnp.int32),
            jax.ShapeDtypeStruct((16,), jnp.float32),
        ),
        mesh=mesh,
        scratch_types=(
            pltpu.VMEM((NE * TPW,), jnp.float32),
            pltpu.VMEM((TPW,), jnp.float32),
            pltpu.VMEM((TPW,), jnp.float32),
            pltpu.VMEM((TPW,), jnp.int32),
            pltpu.VMEM((TPW,), jnp.int32),
            pltpu.VMEM((16,), jnp.float32),
            pltpu.VMEM((NSUB * 16,), jnp.float32),
            pltpu.VMEM_SHARED((NSUB * 16,), jnp.float32),
            pltpu.SemaphoreType.DMA,
        ),
        compiler_params=pltpu.CompilerParams(needs_layout_passes=False),
    )


def kernel(input, W):
    x = input.reshape(N_TOK, HID)
    logit_list = _logits_tc(x, W)
    s1, s2, i1, i2, cnt = _get_sc_route()(*logit_list)
    return (
        jnp.stack([s1, s2], axis=1),
        jnp.stack([i1, i2], axis=1),
        cnt[:NE],
    )


# speedup vs baseline: 1.0427x; 1.0117x over previous
import functools

import jax
import jax.numpy as jnp
from jax import lax
from jax.experimental import pallas as pl
from jax.experimental.pallas import tpu as pltpu
from jax.experimental.pallas import tpu_sc as plsc

N_TOK = 16384
HID = 4096
NE = 8
TOPK = 2

BT = 512


def _logits_body(xa_ref, xb_ref, w_ref, *out_refs):
    dims = (((1,), (1,)), ((), ()))
    blka_t = lax.dot_general(
        xa_ref[...], w_ref[...], dims, preferred_element_type=jnp.float32
    ).T
    blkb_t = lax.dot_general(
        xb_ref[...], w_ref[...], dims, preferred_element_type=jnp.float32
    ).T
    for e in range(NE):
        out_refs[e][pl.ds(0, BT)] = blka_t[e : e + 1, :].reshape(BT)
        out_refs[e][pl.ds(BT, BT)] = blkb_t[e : e + 1, :].reshape(BT)


def _logits_tc(x, w):
    return pl.pallas_call(
        _logits_body,
        grid=(N_TOK // (2 * BT),),
        in_specs=[
            pl.BlockSpec((BT, HID), lambda i: (2 * i, 0)),
            pl.BlockSpec((BT, HID), lambda i: (2 * i + 1, 0)),
            pl.BlockSpec((NE, HID), lambda i: (0, 0)),
        ],
        out_specs=[
            pl.BlockSpec((2 * BT,), lambda i: (i,)) for _ in range(NE)
        ],
        out_shape=[
            jax.ShapeDtypeStruct((N_TOK,), jnp.float32) for _ in range(NE)
        ],
    )(x, x, w)


NSUB = 16
TPW = N_TOK // NSUB
NCH = TPW // 16
UNROLL = 8


def _sc_route_body(*refs):
    logit_hbm = refs[:NE]
    s1_hbm, s2_hbm, i1_hbm, i2_hbm, cnt_hbm = refs[NE : NE + 5]
    lg_v, s1_v, s2_v, i1_v, i2_v, cnt_v, gat_v, part_sh, sem = refs[NE + 5 :]

    wid = lax.axis_index("s")
    base = wid * TPW
    copies = [
        pltpu.make_async_copy(
            logit_hbm[e].at[pl.ds(base, TPW)],
            lg_v.at[pl.ds(e * TPW, TPW)],
            sem,
        )
        for e in range(NE)
    ]
    for c in copies:
        c.start()
    for c in copies:
        c.wait()

    lanes = lax.iota(jnp.int32, 16)

    def chunk(tok0, pk):
        m1 = lg_v[pl.ds(tok0, 16)]
        i1 = jnp.zeros((16,), jnp.int32)
        m2 = jnp.full((16,), -jnp.inf, jnp.float32)
        i2 = jnp.zeros((16,), jnp.int32)
        for e in range(1, NE):
            v = lg_v[pl.ds(e * TPW + tok0, 16)]
            gt1 = v > m1
            gt2 = jnp.logical_and(jnp.logical_not(gt1), v > m2)
            m2 = jnp.where(gt1, m1, jnp.where(gt2, v, m2))
            i2 = jnp.where(gt1, i1, jnp.where(gt2, e, i2))
            m1 = jnp.where(gt1, v, m1)
            i1 = jnp.where(gt1, e, i1)
        d = jnp.exp(m2 - m1)
        r = 1.0 / (1.0 + d)
        sl = pl.ds(tok0, 16)
        s1_v[sl] = r
        s2_v[sl] = d * r
        i1_v[sl] = i1
        i2_v[sl] = i2
        one = jnp.int32(1)
        pk = pk + (one << (i1 * 4)) + (one << (i2 * 4))
        return pk

    def group(g, accs):
        pk = jnp.zeros((16,), jnp.int32)
        for j in range(UNROLL):
            pk = chunk(g * (16 * UNROLL) + j * 16, pk)
        return tuple(
            accs[e] + ((pk >> (4 * e)) & 0xF).astype(jnp.float32)
            for e in range(NE)
        )

    acc0 = tuple(jnp.zeros((16,), jnp.float32) for _ in range(NE))
    accs = lax.fori_loop(0, NCH // UNROLL, group, acc0)

    out_copies = [
        pltpu.make_async_copy(v, h.at[pl.ds(base, TPW)], sem)
        for v, h in (
            (s1_v, s1_hbm), (s2_v, s2_hbm), (i1_v, i1_hbm), (i2_v, i2_hbm)
        )
    ]
    for c in out_copies:
        c.start()
    for c in out_copies:
        c.wait()

    cv = jnp.zeros((16,), jnp.float32)
    for e in range(NE):
        cv = jnp.where(lanes == e, jnp.sum(accs[e]), cv)
    cnt_v[...] = cv
    pltpu.sync_copy(cnt_v, part_sh.at[pl.ds(wid * 16, 16)])
    plsc.subcore_barrier()

    @pl.when(wid == 0)
    def _():
        pltpu.sync_copy(part_sh, gat_v)
        tot = gat_v[pl.ds(0, 16)]
        for w in range(1, NSUB):
            tot = tot + gat_v[pl.ds(w * 16, 16)]
        cnt_v[...] = tot
        pltpu.sync_copy(cnt_v, cnt_hbm)


@functools.lru_cache(maxsize=1)
def _get_sc_route():
    mesh = plsc.VectorSubcoreMesh(
        core_axis_name="c", subcore_axis_name="s",
        num_cores=1, num_subcores=NSUB,
    )
    return pl.kernel(
        _sc_route_body,
        out_type=(
            jax.ShapeDtypeStruct((N_TOK,), jnp.float32),
            jax.ShapeDtypeStruct((N_TOK,), jnp.float32),
            jax.ShapeDtypeStruct((N_TOK,), jnp.int32),
            jax.ShapeDtypeStruct((N_TOK,), jnp.int32),
            jax.ShapeDtypeStruct((16,), jnp.float32),
        ),
        mesh=mesh,
        scratch_types=(
            pltpu.VMEM((NE * TPW,), jnp.float32),
            pltpu.VMEM((TPW,), jnp.float32),
            pltpu.VMEM((TPW,), jnp.float32),
            pltpu.VMEM((TPW,), jnp.int32),
            pltpu.VMEM((TPW,), jnp.int32),
            pltpu.VMEM((16,), jnp.float32),
            pltpu.VMEM((NSUB * 16,), jnp.float32),
            pltpu.VMEM_SHARED((NSUB * 16,), jnp.float32),
            pltpu.SemaphoreType.DMA,
        ),
        compiler_params=pltpu.CompilerParams(needs_layout_passes=False),
    )


def kernel(input, W):
    x = input.reshape(N_TOK, HID)
    logit_list = _logits_tc(x, W)
    s1, s2, i1, i2, cnt = _get_sc_route()(*logit_list)
    return (
        jnp.stack([s1, s2], axis=1),
        jnp.stack([i1, i2], axis=1),
        cnt[:NE],
    )
